# baseline (speedup 1.0000x reference)
import functools

import jax
import jax.numpy as jnp
from jax import lax
from jax.experimental import pallas as pl
from jax.experimental.pallas import tpu as pltpu
from jax.experimental.pallas import tpu_sc as plsc

NUM_NODES = 10000
EMBED = 256
QUART = 64
E = 160000

M_BLK = 400
N_SUB = 16
EDGES_PER_TILE = E // N_SUB
BATCH = 128
N_BATCH = 79
PAD = N_BATCH * BATCH - EDGES_PER_TILE
ACC_ROWS = NUM_NODES + 8
SLAB = 624
ZCH = 104


def _quarter_specs(blk):
    return [pl.BlockSpec((blk, QUART), lambda i: (i, 0)) for _ in range(4)]


def _k1_body(nid_ref, wemb_ref, bemb_ref, w1_ref, dinv_ref,
             h0_ref, y0_ref, y1_ref, y2_ref, y3_ref):
    h0 = jnp.dot(nid_ref[...], wemb_ref[...],
                 preferred_element_type=jnp.float32) + bemb_ref[...]
    h0_ref[...] = h0
    y = jnp.dot(h0, w1_ref[...], preferred_element_type=jnp.float32) * dinv_ref[...]
    y0_ref[...] = y[:, 0 * QUART:1 * QUART]
    y1_ref[...] = y[:, 1 * QUART:2 * QUART]
    y2_ref[...] = y[:, 2 * QUART:3 * QUART]
    y3_ref[...] = y[:, 3 * QUART:4 * QUART]


def _k1(node_ids, w_emb, b_emb2d, w1, dinv2d):
    grid = (NUM_NODES // M_BLK,)
    q = jax.ShapeDtypeStruct((NUM_NODES, QUART), jnp.float32)
    return pl.pallas_call(
        _k1_body,
        grid=grid,
        in_specs=[
            pl.BlockSpec((M_BLK, NUM_NODES), lambda i: (i, 0)),
            pl.BlockSpec((NUM_NODES, EMBED), lambda i: (0, 0)),
            pl.BlockSpec((1, EMBED), lambda i: (0, 0)),
            pl.BlockSpec((EMBED, EMBED), lambda i: (0, 0)),
            pl.BlockSpec((M_BLK, 1), lambda i: (i, 0)),
        ],
        out_specs=[pl.BlockSpec((M_BLK, EMBED), lambda i: (i, 0))]
        + _quarter_specs(M_BLK),
        out_shape=[jax.ShapeDtypeStruct((NUM_NODES, EMBED), jnp.float32),
                   q, q, q, q],
    )(node_ids, w_emb, b_emb2d, w1, dinv2d)


def _k3_body(a0_ref, a1_ref, a2_ref, a3_ref, y0_ref, y1_ref, y2_ref, y3_ref,
             dinv_ref, b1_ref, w2_ref, o0_ref, o1_ref, o2_ref, o3_ref):
    d = dinv_ref[...]
    parts = [(a_ref[...] + y_ref[...]) * d
             for a_ref, y_ref in ((a0_ref, y0_ref), (a1_ref, y1_ref),
                                  (a2_ref, y2_ref), (a3_ref, y3_ref))]
    h1 = jax.nn.relu(jnp.concatenate(parts, axis=1) + b1_ref[...])
    y2 = jnp.dot(h1, w2_ref[...], preferred_element_type=jnp.float32) * d
    o0_ref[...] = y2[:, 0 * QUART:1 * QUART]
    o1_ref[...] = y2[:, 1 * QUART:2 * QUART]
    o2_ref[...] = y2[:, 2 * QUART:3 * QUART]
    o3_ref[...] = y2[:, 3 * QUART:4 * QUART]


def _k3(aggs, ys, dinv2d, b1_2d, w2):
    blk = 1000
    grid = (NUM_NODES // blk,)
    q = jax.ShapeDtypeStruct((NUM_NODES, QUART), jnp.float32)
    return pl.pallas_call(
        _k3_body,
        grid=grid,
        in_specs=_quarter_specs(blk) + _quarter_specs(blk) + [
            pl.BlockSpec((blk, 1), lambda i: (i, 0)),
            pl.BlockSpec((1, EMBED), lambda i: (0, 0)),
            pl.BlockSpec((EMBED, EMBED), lambda i: (0, 0)),
        ],
        out_specs=_quarter_specs(blk),
        out_shape=[q, q, q, q],
    )(*aggs, *ys, dinv2d, b1_2d, w2)


def _k5_body(a0_ref, a1_ref, a2_ref, a3_ref, y0_ref, y1_ref, y2_ref, y3_ref,
             dinv_ref, b2_ref, h0_ref, out_ref):
    d = dinv_ref[...]
    parts = [(a_ref[...] + y_ref[...]) * d
             for a_ref, y_ref in ((a0_ref, y0_ref), (a1_ref, y1_ref),
                                  (a2_ref, y2_ref), (a3_ref, y3_ref))]
    out_ref[...] = jnp.concatenate(parts, axis=1) + b2_ref[...] + h0_ref[...]


def _k5(aggs, ys, dinv2d, b2_2d, h0):
    blk = 1000
    grid = (NUM_NODES // blk,)
    return pl.pallas_call(
        _k5_body,
        grid=grid,
        in_specs=_quarter_specs(blk) + _quarter_specs(blk) + [
            pl.BlockSpec((blk, 1), lambda i: (i, 0)),
            pl.BlockSpec((1, EMBED), lambda i: (0, 0)),
            pl.BlockSpec((blk, EMBED), lambda i: (i, 0)),
        ],
        out_specs=pl.BlockSpec((blk, EMBED), lambda i: (i, 0)),
        out_shape=jax.ShapeDtypeStruct((NUM_NODES, EMBED), jnp.float32),
    )(*aggs, *ys, dinv2d, b2_2d, h0)


def _sc_agg_body(yq0_hbm, yq1_hbm, yq2_hbm, yq3_hbm, src_hbm, dst_hbm,
                 o0_hbm, o1_hbm, o2_hbm, o3_hbm,
                 idx_v, rr_v, g0, g1, s0, s1, agg_sh):
    c = lax.axis_index("c")
    s = lax.axis_index("s")
    base = s * SLAB

    pltpu.sync_copy(src_hbm.at[s], idx_v.at[0])
    pltpu.sync_copy(dst_hbm.at[s], idx_v.at[1])

    def _zero_agg():
        def _zero_row(i, carry):
            for j in range(QUART // 16):
                rr_v[0, i, pl.ds(j * 16, 16)] = jnp.zeros((16,), jnp.float32)
            return carry
        lax.fori_loop(0, ZCH, _zero_row, 0)
        for i in range(SLAB // ZCH):
            pltpu.sync_copy(rr_v.at[0, pl.ds(0, ZCH), :],
                            agg_sh.at[pl.ds(base + i * ZCH, ZCH), :])

        @pl.when(s == N_SUB - 1)
        def _zero_tail():
            pltpu.sync_copy(rr_v.at[0, pl.ds(0, 24), :],
                            agg_sh.at[pl.ds(N_SUB * SLAB, 24), :])

    def _run(y_ref):
        rbuf = rr_v.at[0]
        rbuf1 = rr_v.at[1]

        def gather(b, r, sem):
            pltpu.async_copy(y_ref.at[idx_v.at[0, b]], r, sem)

        def scatter(b, r, sem):
            pltpu.async_copy(r, agg_sh.at[idx_v.at[1, b]], sem, add=True)

        def wait_gather(r, sem):
            pltpu.make_async_copy(y_ref.at[idx_v.at[0, 0]], r, sem).wait()

        def wait_scatter(r, sem):
            pltpu.make_async_copy(r, agg_sh.at[idx_v.at[1, 0]], sem).wait()

        gather(0, rbuf, g0)
        gather(1, rbuf1, g1)

        def body(p, carry):
            b = 2 * p
            wait_gather(rbuf, g0)
            scatter(b, rbuf, s0)
            wait_gather(rbuf1, g1)
            scatter(b + 1, rbuf1, s1)

            @pl.when(b + 2 < N_BATCH)
            def _next0():
                wait_scatter(rbuf, s0)
                gather(b + 2, rbuf, g0)

            @pl.when(b + 3 < N_BATCH)
            def _next1():
                wait_scatter(rbuf1, s1)
                gather(b + 3, rbuf1, g1)
            return carry
        lax.fori_loop(0, N_BATCH // 2, body, 0)
        if N_BATCH % 2:
            wait_gather(rbuf, g0)
            scatter(N_BATCH - 1, rbuf, s0)
        wait_scatter(rbuf, s0)
        wait_scatter(rbuf1, s1)

    def _writeback(out_ref):
        pltpu.sync_copy(agg_sh.at[pl.ds(base, SLAB), :],
                        out_ref.at[pl.ds(base, SLAB), :])

        @pl.when(s == N_SUB - 1)
        def _tail():
            pltpu.sync_copy(agg_sh.at[pl.ds(N_SUB * SLAB, 16), :],
                            out_ref.at[pl.ds(N_SUB * SLAB, 16), :])

    for k in range(2):
        _zero_agg()
        plsc.subcore_barrier()
        pl.when(c == 0)(lambda: _run((yq0_hbm, yq1_hbm)[k]))
        pl.when(c == 1)(lambda: _run((yq2_hbm, yq3_hbm)[k]))
        plsc.subcore_barrier()
        pl.when(c == 0)(lambda: _writeback((o0_hbm, o1_hbm)[k]))
        pl.when(c == 1)(lambda: _writeback((o2_hbm, o3_hbm)[k]))


def _sc_agg(ys, src3, dst3):
    mesh = plsc.VectorSubcoreMesh(core_axis_name="c", subcore_axis_name="s")
    q = jax.ShapeDtypeStruct((NUM_NODES, QUART), jnp.float32)
    return pl.kernel(
        _sc_agg_body,
        out_type=[q, q, q, q],
        mesh=mesh,
        compiler_params=pltpu.CompilerParams(use_tc_tiling_on_sc=False),
        scratch_types=[
            pltpu.VMEM((2, N_BATCH, BATCH), jnp.int32),
            pltpu.VMEM((2, BATCH, QUART), jnp.float32),
            pltpu.SemaphoreType.DMA,
            pltpu.SemaphoreType.DMA,
            pltpu.SemaphoreType.DMA,
            pltpu.SemaphoreType.DMA,
            pltpu.VMEM_SHARED((ACC_ROWS, QUART), jnp.float32),
        ],
    )(*ys, src3, dst3)


def kernel(node_ids, edge_index, W_emb, b_emb, W1, b1, W2, b2):
    src = edge_index[0]
    dst = edge_index[1]
    pad_src = (jnp.arange(PAD, dtype=jnp.int32) * 89) % NUM_NODES
    pad_dst = NUM_NODES + (jnp.arange(PAD, dtype=jnp.int32) % 8)
    src3 = jnp.concatenate(
        [src.reshape(N_SUB, EDGES_PER_TILE),
         jnp.broadcast_to(pad_src, (N_SUB, PAD))], axis=1
    ).reshape(N_SUB, N_BATCH, BATCH)
    dst3 = jnp.concatenate(
        [dst.reshape(N_SUB, EDGES_PER_TILE),
         jnp.broadcast_to(pad_dst, (N_SUB, PAD))], axis=1
    ).reshape(N_SUB, N_BATCH, BATCH)

    deg = jnp.ones((NUM_NODES,), jnp.float32).at[dst].add(1.0)
    dinv2d = lax.rsqrt(deg)[:, None]

    h0, *y1q = _k1(node_ids, W_emb, b_emb[None, :], W1, dinv2d)
    a1q = _sc_agg(y1q, src3, dst3)
    y2q = _k3(a1q, y1q, dinv2d, b1[None, :], W2)
    a2q = _sc_agg(y2q, src3, dst3)
    return _k5(a2q, y2q, dinv2d, b2[None, :], h0)


# speedup vs baseline: 1.0551x; 1.0551x over previous
import functools

import jax
import jax.numpy as jnp
from jax import lax
from jax.experimental import pallas as pl
from jax.experimental.pallas import tpu as pltpu
from jax.experimental.pallas import tpu_sc as plsc

NUM_NODES = 10000
EMBED = 256
QUART = 64
E = 160000

M_BLK = 400
N_SUB = 16
EDGES_PER_TILE = E // N_SUB
BATCH = 512
N_BATCH = 20
PAD = N_BATCH * BATCH - EDGES_PER_TILE
ACC_ROWS = NUM_NODES + 8
SLAB = 624
ZCH = 104


def _quarter_specs(blk):
    return [pl.BlockSpec((blk, QUART), lambda i: (i, 0)) for _ in range(4)]


def _k1_body(nid_ref, wemb_ref, bemb_ref, w1_ref, dinv_ref,
             h0_ref, y0_ref, y1_ref, y2_ref, y3_ref):
    h0 = jnp.dot(nid_ref[...], wemb_ref[...],
                 preferred_element_type=jnp.float32) + bemb_ref[...]
    h0_ref[...] = h0
    y = jnp.dot(h0, w1_ref[...], preferred_element_type=jnp.float32) * dinv_ref[...]
    y0_ref[...] = y[:, 0 * QUART:1 * QUART]
    y1_ref[...] = y[:, 1 * QUART:2 * QUART]
    y2_ref[...] = y[:, 2 * QUART:3 * QUART]
    y3_ref[...] = y[:, 3 * QUART:4 * QUART]


def _k1(node_ids, w_emb, b_emb2d, w1, dinv2d):
    grid = (NUM_NODES // M_BLK,)
    q = jax.ShapeDtypeStruct((NUM_NODES, QUART), jnp.float32)
    return pl.pallas_call(
        _k1_body,
        grid=grid,
        in_specs=[
            pl.BlockSpec((M_BLK, NUM_NODES), lambda i: (i, 0)),
            pl.BlockSpec((NUM_NODES, EMBED), lambda i: (0, 0)),
            pl.BlockSpec((1, EMBED), lambda i: (0, 0)),
            pl.BlockSpec((EMBED, EMBED), lambda i: (0, 0)),
            pl.BlockSpec((M_BLK, 1), lambda i: (i, 0)),
        ],
        out_specs=[pl.BlockSpec((M_BLK, EMBED), lambda i: (i, 0))]
        + _quarter_specs(M_BLK),
        out_shape=[jax.ShapeDtypeStruct((NUM_NODES, EMBED), jnp.float32),
                   q, q, q, q],
    )(node_ids, w_emb, b_emb2d, w1, dinv2d)


def _k3_body(a0_ref, a1_ref, a2_ref, a3_ref, y0_ref, y1_ref, y2_ref, y3_ref,
             dinv_ref, b1_ref, w2_ref, o0_ref, o1_ref, o2_ref, o3_ref):
    d = dinv_ref[...]
    parts = [(a_ref[...] + y_ref[...]) * d
             for a_ref, y_ref in ((a0_ref, y0_ref), (a1_ref, y1_ref),
                                  (a2_ref, y2_ref), (a3_ref, y3_ref))]
    h1 = jax.nn.relu(jnp.concatenate(parts, axis=1) + b1_ref[...])
    y2 = jnp.dot(h1, w2_ref[...], preferred_element_type=jnp.float32) * d
    o0_ref[...] = y2[:, 0 * QUART:1 * QUART]
    o1_ref[...] = y2[:, 1 * QUART:2 * QUART]
    o2_ref[...] = y2[:, 2 * QUART:3 * QUART]
    o3_ref[...] = y2[:, 3 * QUART:4 * QUART]


def _k3(aggs, ys, dinv2d, b1_2d, w2):
    blk = 1000
    grid = (NUM_NODES // blk,)
    q = jax.ShapeDtypeStruct((NUM_NODES, QUART), jnp.float32)
    return pl.pallas_call(
        _k3_body,
        grid=grid,
        in_specs=_quarter_specs(blk) + _quarter_specs(blk) + [
            pl.BlockSpec((blk, 1), lambda i: (i, 0)),
            pl.BlockSpec((1, EMBED), lambda i: (0, 0)),
            pl.BlockSpec((EMBED, EMBED), lambda i: (0, 0)),
        ],
        out_specs=_quarter_specs(blk),
        out_shape=[q, q, q, q],
    )(*aggs, *ys, dinv2d, b1_2d, w2)


def _k5_body(a0_ref, a1_ref, a2_ref, a3_ref, y0_ref, y1_ref, y2_ref, y3_ref,
             dinv_ref, b2_ref, h0_ref, out_ref):
    d = dinv_ref[...]
    parts = [(a_ref[...] + y_ref[...]) * d
             for a_ref, y_ref in ((a0_ref, y0_ref), (a1_ref, y1_ref),
                                  (a2_ref, y2_ref), (a3_ref, y3_ref))]
    out_ref[...] = jnp.concatenate(parts, axis=1) + b2_ref[...] + h0_ref[...]


def _k5(aggs, ys, dinv2d, b2_2d, h0):
    blk = 1000
    grid = (NUM_NODES // blk,)
    return pl.pallas_call(
        _k5_body,
        grid=grid,
        in_specs=_quarter_specs(blk) + _quarter_specs(blk) + [
            pl.BlockSpec((blk, 1), lambda i: (i, 0)),
            pl.BlockSpec((1, EMBED), lambda i: (0, 0)),
            pl.BlockSpec((blk, EMBED), lambda i: (i, 0)),
        ],
        out_specs=pl.BlockSpec((blk, EMBED), lambda i: (i, 0)),
        out_shape=jax.ShapeDtypeStruct((NUM_NODES, EMBED), jnp.float32),
    )(*aggs, *ys, dinv2d, b2_2d, h0)


def _sc_agg_body(yq0_hbm, yq1_hbm, yq2_hbm, yq3_hbm, src_hbm, dst_hbm,
                 o0_hbm, o1_hbm, o2_hbm, o3_hbm,
                 idx_v, rr_v, g0, g1, s0, s1, agg_sh):
    c = lax.axis_index("c")
    s = lax.axis_index("s")
    base = s * SLAB

    pltpu.sync_copy(src_hbm.at[s], idx_v.at[0])
    pltpu.sync_copy(dst_hbm.at[s], idx_v.at[1])

    def _zero_agg():
        def _zero_row(i, carry):
            for j in range(QUART // 16):
                rr_v[0, i, pl.ds(j * 16, 16)] = jnp.zeros((16,), jnp.float32)
            return carry
        lax.fori_loop(0, ZCH, _zero_row, 0)
        for i in range(SLAB // ZCH):
            pltpu.sync_copy(rr_v.at[0, pl.ds(0, ZCH), :],
                            agg_sh.at[pl.ds(base + i * ZCH, ZCH), :])

        @pl.when(s == N_SUB - 1)
        def _zero_tail():
            pltpu.sync_copy(rr_v.at[0, pl.ds(0, 24), :],
                            agg_sh.at[pl.ds(N_SUB * SLAB, 24), :])

    def _run(y_ref):
        rbuf = rr_v.at[0]
        rbuf1 = rr_v.at[1]

        def gather(b, r, sem):
            pltpu.async_copy(y_ref.at[idx_v.at[0, b]], r, sem)

        def scatter(b, r, sem):
            pltpu.async_copy(r, agg_sh.at[idx_v.at[1, b]], sem, add=True)

        def wait_gather(r, sem):
            pltpu.make_async_copy(y_ref.at[idx_v.at[0, 0]], r, sem).wait()

        def wait_scatter(r, sem):
            pltpu.make_async_copy(r, agg_sh.at[idx_v.at[1, 0]], sem).wait()

        gather(0, rbuf, g0)
        gather(1, rbuf1, g1)

        def body(p, carry):
            b = 2 * p
            wait_gather(rbuf, g0)
            scatter(b, rbuf, s0)
            wait_gather(rbuf1, g1)
            scatter(b + 1, rbuf1, s1)

            @pl.when(b + 2 < N_BATCH)
            def _next0():
                wait_scatter(rbuf, s0)
                gather(b + 2, rbuf, g0)

            @pl.when(b + 3 < N_BATCH)
            def _next1():
                wait_scatter(rbuf1, s1)
                gather(b + 3, rbuf1, g1)
            return carry
        lax.fori_loop(0, N_BATCH // 2, body, 0)
        if N_BATCH % 2:
            wait_gather(rbuf, g0)
            scatter(N_BATCH - 1, rbuf, s0)
        wait_scatter(rbuf, s0)
        wait_scatter(rbuf1, s1)

    def _writeback(out_ref):
        pltpu.sync_copy(agg_sh.at[pl.ds(base, SLAB), :],
                        out_ref.at[pl.ds(base, SLAB), :])

        @pl.when(s == N_SUB - 1)
        def _tail():
            pltpu.sync_copy(agg_sh.at[pl.ds(N_SUB * SLAB, 16), :],
                            out_ref.at[pl.ds(N_SUB * SLAB, 16), :])

    for k in range(2):
        _zero_agg()
        plsc.subcore_barrier()
        pl.when(c == 0)(lambda: _run((yq0_hbm, yq1_hbm)[k]))
        pl.when(c == 1)(lambda: _run((yq2_hbm, yq3_hbm)[k]))
        plsc.subcore_barrier()
        pl.when(c == 0)(lambda: _writeback((o0_hbm, o1_hbm)[k]))
        pl.when(c == 1)(lambda: _writeback((o2_hbm, o3_hbm)[k]))


def _sc_agg(ys, src3, dst3):
    mesh = plsc.VectorSubcoreMesh(core_axis_name="c", subcore_axis_name="s")
    q = jax.ShapeDtypeStruct((NUM_NODES, QUART), jnp.float32)
    return pl.kernel(
        _sc_agg_body,
        out_type=[q, q, q, q],
        mesh=mesh,
        compiler_params=pltpu.CompilerParams(use_tc_tiling_on_sc=False),
        scratch_types=[
            pltpu.VMEM((2, N_BATCH, BATCH), jnp.int32),
            pltpu.VMEM((2, BATCH, QUART), jnp.float32),
            pltpu.SemaphoreType.DMA,
            pltpu.SemaphoreType.DMA,
            pltpu.SemaphoreType.DMA,
            pltpu.SemaphoreType.DMA,
            pltpu.VMEM_SHARED((ACC_ROWS, QUART), jnp.float32),
        ],
    )(*ys, src3, dst3)


def kernel(node_ids, edge_index, W_emb, b_emb, W1, b1, W2, b2):
    src = edge_index[0]
    dst = edge_index[1]
    pad_src = (jnp.arange(PAD, dtype=jnp.int32) * 89) % NUM_NODES
    pad_dst = NUM_NODES + (jnp.arange(PAD, dtype=jnp.int32) % 8)
    src3 = jnp.concatenate(
        [src.reshape(N_SUB, EDGES_PER_TILE),
         jnp.broadcast_to(pad_src, (N_SUB, PAD))], axis=1
    ).reshape(N_SUB, N_BATCH, BATCH)
    dst3 = jnp.concatenate(
        [dst.reshape(N_SUB, EDGES_PER_TILE),
         jnp.broadcast_to(pad_dst, (N_SUB, PAD))], axis=1
    ).reshape(N_SUB, N_BATCH, BATCH)

    deg = jnp.ones((NUM_NODES,), jnp.float32).at[dst].add(1.0)
    dinv2d = lax.rsqrt(deg)[:, None]

    h0, *y1q = _k1(node_ids, W_emb, b_emb[None, :], W1, dinv2d)
    a1q = _sc_agg(y1q, src3, dst3)
    y2q = _k3(a1q, y1q, dinv2d, b1[None, :], W2)
    a2q = _sc_agg(y2q, src3, dst3)
    return _k5(a2q, y2q, dinv2d, b2[None, :], h0)


# speedup vs baseline: 1.3006x; 1.2327x over previous
import functools

import jax
import jax.numpy as jnp
from jax import lax
from jax.experimental import pallas as pl
from jax.experimental.pallas import tpu as pltpu
from jax.experimental.pallas import tpu_sc as plsc

NUM_NODES = 10000
EMBED = 256
HALF = 128
E = 160000

M_BLK = 400
N_SUB = 16
EDGES_PER_TILE = E // N_SUB
BATCH = 512
N_BATCH = 20
PAD = N_BATCH * BATCH - EDGES_PER_TILE
ACC_ROWS = NUM_NODES + 8
SLAB = 624
ZCH = 104
BF = jnp.bfloat16


def _k1_body(nid_ref, wemb_ref, bemb_ref, w1_ref, dinv_ref,
             h0_ref, y0_ref, y1_ref):
    h0 = jnp.dot(nid_ref[...], wemb_ref[...],
                 preferred_element_type=jnp.float32) + bemb_ref[...]
    h0_ref[...] = h0
    y = jnp.dot(h0, w1_ref[...], preferred_element_type=jnp.float32) * dinv_ref[...]
    y0_ref[...] = y[:, :HALF].astype(BF)
    y1_ref[...] = y[:, HALF:].astype(BF)


def _k1(node_ids, w_emb, b_emb2d, w1, dinv2d):
    grid = (NUM_NODES // M_BLK,)
    yh = jax.ShapeDtypeStruct((NUM_NODES, HALF), BF)
    return pl.pallas_call(
        _k1_body,
        grid=grid,
        in_specs=[
            pl.BlockSpec((M_BLK, NUM_NODES), lambda i: (i, 0)),
            pl.BlockSpec((NUM_NODES, EMBED), lambda i: (0, 0)),
            pl.BlockSpec((1, EMBED), lambda i: (0, 0)),
            pl.BlockSpec((EMBED, EMBED), lambda i: (0, 0)),
            pl.BlockSpec((M_BLK, 1), lambda i: (i, 0)),
        ],
        out_specs=[
            pl.BlockSpec((M_BLK, EMBED), lambda i: (i, 0)),
            pl.BlockSpec((M_BLK, HALF), lambda i: (i, 0)),
            pl.BlockSpec((M_BLK, HALF), lambda i: (i, 0)),
        ],
        out_shape=[jax.ShapeDtypeStruct((NUM_NODES, EMBED), jnp.float32),
                   yh, yh],
    )(node_ids, w_emb, b_emb2d, w1, dinv2d)


def _k3_body(a0_ref, a1_ref, y0_ref, y1_ref, dinv_ref, b1_ref, w2_ref,
             o0_ref, o1_ref):
    d = dinv_ref[...]
    h1a = (a0_ref[...].astype(jnp.float32) + y0_ref[...].astype(jnp.float32)) * d
    h1b = (a1_ref[...].astype(jnp.float32) + y1_ref[...].astype(jnp.float32)) * d
    h1 = jax.nn.relu(jnp.concatenate([h1a, h1b], axis=1) + b1_ref[...])
    y2 = jnp.dot(h1, w2_ref[...], preferred_element_type=jnp.float32) * d
    o0_ref[...] = y2[:, :HALF].astype(BF)
    o1_ref[...] = y2[:, HALF:].astype(BF)


def _k3(aggs, ys, dinv2d, b1_2d, w2):
    blk = 1000
    grid = (NUM_NODES // blk,)
    half_in = pl.BlockSpec((blk, HALF), lambda i: (i, 0))
    yh = jax.ShapeDtypeStruct((NUM_NODES, HALF), BF)
    return pl.pallas_call(
        _k3_body,
        grid=grid,
        in_specs=[half_in, half_in, half_in, half_in,
                  pl.BlockSpec((blk, 1), lambda i: (i, 0)),
                  pl.BlockSpec((1, EMBED), lambda i: (0, 0)),
                  pl.BlockSpec((EMBED, EMBED), lambda i: (0, 0))],
        out_specs=[half_in, half_in],
        out_shape=[yh, yh],
    )(*aggs, *ys, dinv2d, b1_2d, w2)


def _k5_body(a0_ref, a1_ref, y0_ref, y1_ref, dinv_ref, b2_ref, h0_ref,
             out_ref):
    d = dinv_ref[...]
    oa = (a0_ref[...].astype(jnp.float32) + y0_ref[...].astype(jnp.float32)) * d
    ob = (a1_ref[...].astype(jnp.float32) + y1_ref[...].astype(jnp.float32)) * d
    out_ref[...] = jnp.concatenate([oa, ob], axis=1) + b2_ref[...] + h0_ref[...]


def _k5(aggs, ys, dinv2d, b2_2d, h0):
    blk = 1000
    grid = (NUM_NODES // blk,)
    half_in = pl.BlockSpec((blk, HALF), lambda i: (i, 0))
    return pl.pallas_call(
        _k5_body,
        grid=grid,
        in_specs=[half_in, half_in, half_in, half_in,
                  pl.BlockSpec((blk, 1), lambda i: (i, 0)),
                  pl.BlockSpec((1, EMBED), lambda i: (0, 0)),
                  pl.BlockSpec((blk, EMBED), lambda i: (i, 0))],
        out_specs=pl.BlockSpec((blk, EMBED), lambda i: (i, 0)),
        out_shape=jax.ShapeDtypeStruct((NUM_NODES, EMBED), jnp.float32),
    )(*aggs, *ys, dinv2d, b2_2d, h0)


def _sc_agg_body(y0_hbm, y1_hbm, src_hbm, dst_hbm, out0_hbm, out1_hbm,
                 idx_v, rr_v, g0, g1, s0, s1, agg_sh):
    c = lax.axis_index("c")
    s = lax.axis_index("s")
    base = s * SLAB

    pltpu.sync_copy(src_hbm.at[s], idx_v.at[0])
    pltpu.sync_copy(dst_hbm.at[s], idx_v.at[1])

    def _zero_row(i, carry):
        for j in range(HALF // 32):
            rr_v[0, i, pl.ds(j * 32, 32)] = jnp.zeros((32,), BF)
        return carry
    lax.fori_loop(0, ZCH, _zero_row, 0)
    for i in range(SLAB // ZCH):
        pltpu.sync_copy(rr_v.at[0, pl.ds(0, ZCH), :],
                        agg_sh.at[pl.ds(base + i * ZCH, ZCH), :])

    @pl.when(s == N_SUB - 1)
    def _zero_tail():
        pltpu.sync_copy(rr_v.at[0, pl.ds(0, 24), :],
                        agg_sh.at[pl.ds(N_SUB * SLAB, 24), :])

    plsc.subcore_barrier()

    def _run(y_ref):
        rbuf = rr_v.at[0]
        rbuf1 = rr_v.at[1]

        def gather(b, r, sem):
            pltpu.async_copy(y_ref.at[idx_v.at[0, b]], r, sem)

        def scatter(b, r, sem):
            pltpu.async_copy(r, agg_sh.at[idx_v.at[1, b]], sem, add=True)

        def wait_gather(r, sem):
            pltpu.make_async_copy(y_ref.at[idx_v.at[0, 0]], r, sem).wait()

        def wait_scatter(r, sem):
            pltpu.make_async_copy(r, agg_sh.at[idx_v.at[1, 0]], sem).wait()

        gather(0, rbuf, g0)
        gather(1, rbuf1, g1)

        def body(p, carry):
            b = 2 * p
            wait_gather(rbuf, g0)
            scatter(b, rbuf, s0)
            wait_gather(rbuf1, g1)
            scatter(b + 1, rbuf1, s1)

            @pl.when(b + 2 < N_BATCH)
            def _next0():
                wait_scatter(rbuf, s0)
                gather(b + 2, rbuf, g0)

            @pl.when(b + 3 < N_BATCH)
            def _next1():
                wait_scatter(rbuf1, s1)
                gather(b + 3, rbuf1, g1)
            return carry
        lax.fori_loop(0, N_BATCH // 2, body, 0)
        if N_BATCH % 2:
            wait_gather(rbuf, g0)
            scatter(N_BATCH - 1, rbuf, s0)
        wait_scatter(rbuf, s0)
        wait_scatter(rbuf1, s1)

    pl.when(c == 0)(lambda: _run(y0_hbm))
    pl.when(c == 1)(lambda: _run(y1_hbm))
    plsc.subcore_barrier()

    def _writeback(out_ref):
        pltpu.sync_copy(agg_sh.at[pl.ds(base, SLAB), :],
                        out_ref.at[pl.ds(base, SLAB), :])

        @pl.when(s == N_SUB - 1)
        def _tail():
            pltpu.sync_copy(agg_sh.at[pl.ds(N_SUB * SLAB, 16), :],
                            out_ref.at[pl.ds(N_SUB * SLAB, 16), :])

    pl.when(c == 0)(lambda: _writeback(out0_hbm))
    pl.when(c == 1)(lambda: _writeback(out1_hbm))


def _sc_agg(ys, src3, dst3):
    mesh = plsc.VectorSubcoreMesh(core_axis_name="c", subcore_axis_name="s")
    yh = jax.ShapeDtypeStruct((NUM_NODES, HALF), BF)
    return pl.kernel(
        _sc_agg_body,
        out_type=[yh, yh],
        mesh=mesh,
        compiler_params=pltpu.CompilerParams(use_tc_tiling_on_sc=False),
        scratch_types=[
            pltpu.VMEM((2, N_BATCH, BATCH), jnp.int32),
            pltpu.VMEM((2, BATCH, HALF), BF),
            pltpu.SemaphoreType.DMA,
            pltpu.SemaphoreType.DMA,
            pltpu.SemaphoreType.DMA,
            pltpu.SemaphoreType.DMA,
            pltpu.VMEM_SHARED((ACC_ROWS, HALF), BF),
        ],
    )(*ys, src3, dst3)


def kernel(node_ids, edge_index, W_emb, b_emb, W1, b1, W2, b2):
    src = edge_index[0]
    dst = edge_index[1]
    pad_src = (jnp.arange(PAD, dtype=jnp.int32) * 89) % NUM_NODES
    pad_dst = NUM_NODES + (jnp.arange(PAD, dtype=jnp.int32) % 8)
    src3 = jnp.concatenate(
        [src.reshape(N_SUB, EDGES_PER_TILE),
         jnp.broadcast_to(pad_src, (N_SUB, PAD))], axis=1
    ).reshape(N_SUB, N_BATCH, BATCH)
    dst3 = jnp.concatenate(
        [dst.reshape(N_SUB, EDGES_PER_TILE),
         jnp.broadcast_to(pad_dst, (N_SUB, PAD))], axis=1
    ).reshape(N_SUB, N_BATCH, BATCH)

    deg = jnp.ones((NUM_NODES,), jnp.float32).at[dst].add(1.0)
    dinv2d = lax.rsqrt(deg)[:, None]

    h0, y1_0, y1_1 = _k1(node_ids, W_emb, b_emb[None, :], W1, dinv2d)
    a1 = _sc_agg((y1_0, y1_1), src3, dst3)
    y2 = _k3(a1, (y1_0, y1_1), dinv2d, b1[None, :], W2)
    a2 = _sc_agg(y2, src3, dst3)
    return _k5(a2, y2, dinv2d, b2[None, :], h0)


# speedup vs baseline: 1.3865x; 1.0661x over previous
import functools

import jax
import jax.numpy as jnp
from jax import lax
from jax.experimental import pallas as pl
from jax.experimental.pallas import tpu as pltpu
from jax.experimental.pallas import tpu_sc as plsc

NUM_NODES = 10000
EMBED = 256
HALF = 128
E = 160000

M_BLK = 400
N_SUB = 16
EDGES_PER_TILE = E // N_SUB
BATCH = 512
N_BATCH = 20
PAD = N_BATCH * BATCH - EDGES_PER_TILE
ACC_ROWS = NUM_NODES + 8
SLAB = 624
ZCH = 104
BF = jnp.bfloat16


def _k1a_body(nid_ref, wemb_ref, bemb_ref, h0_ref):
    h0_ref[...] = jnp.dot(nid_ref[...], wemb_ref[...],
                          preferred_element_type=jnp.float32) + bemb_ref[...]


def _k1a(node_ids, w_emb, b_emb2d):
    grid = (NUM_NODES // M_BLK,)
    return pl.pallas_call(
        _k1a_body,
        grid=grid,
        in_specs=[
            pl.BlockSpec((M_BLK, NUM_NODES), lambda i: (i, 0)),
            pl.BlockSpec((NUM_NODES, EMBED), lambda i: (0, 0)),
            pl.BlockSpec((1, EMBED), lambda i: (0, 0)),
        ],
        out_specs=pl.BlockSpec((M_BLK, EMBED), lambda i: (i, 0)),
        out_shape=jax.ShapeDtypeStruct((NUM_NODES, EMBED), jnp.float32),
    )(node_ids, w_emb, b_emb2d)


def _k1b_body(h0_ref, w1_ref, dinv_ref, y0_ref, y1_ref):
    y = jnp.dot(h0_ref[...], w1_ref[...],
                preferred_element_type=jnp.float32) * dinv_ref[...]
    y0_ref[...] = y[:, :HALF].astype(BF)
    y1_ref[...] = y[:, HALF:].astype(BF)


def _k1b(h0, w1, dinv2d):
    blk = 1000
    grid = (NUM_NODES // blk,)
    yh = jax.ShapeDtypeStruct((NUM_NODES, HALF), BF)
    return pl.pallas_call(
        _k1b_body,
        grid=grid,
        in_specs=[
            pl.BlockSpec((blk, EMBED), lambda i: (i, 0)),
            pl.BlockSpec((EMBED, EMBED), lambda i: (0, 0)),
            pl.BlockSpec((blk, 1), lambda i: (i, 0)),
        ],
        out_specs=[
            pl.BlockSpec((blk, HALF), lambda i: (i, 0)),
            pl.BlockSpec((blk, HALF), lambda i: (i, 0)),
        ],
        out_shape=[yh, yh],
    )(h0, w1, dinv2d)


def _k3_body(a0_ref, a1_ref, y0_ref, y1_ref, dinv_ref, b1_ref, w2_ref,
             o0_ref, o1_ref):
    d = dinv_ref[...]
    h1a = (a0_ref[...].astype(jnp.float32) + y0_ref[...].astype(jnp.float32)) * d
    h1b = (a1_ref[...].astype(jnp.float32) + y1_ref[...].astype(jnp.float32)) * d
    h1 = jax.nn.relu(jnp.concatenate([h1a, h1b], axis=1) + b1_ref[...])
    y2 = jnp.dot(h1, w2_ref[...], preferred_element_type=jnp.float32) * d
    o0_ref[...] = y2[:, :HALF].astype(BF)
    o1_ref[...] = y2[:, HALF:].astype(BF)


def _k3(aggs, ys, dinv2d, b1_2d, w2):
    blk = 1000
    grid = (NUM_NODES // blk,)
    half_in = pl.BlockSpec((blk, HALF), lambda i: (i, 0))
    yh = jax.ShapeDtypeStruct((NUM_NODES, HALF), BF)
    return pl.pallas_call(
        _k3_body,
        grid=grid,
        in_specs=[half_in, half_in, half_in, half_in,
                  pl.BlockSpec((blk, 1), lambda i: (i, 0)),
                  pl.BlockSpec((1, EMBED), lambda i: (0, 0)),
                  pl.BlockSpec((EMBED, EMBED), lambda i: (0, 0))],
        out_specs=[half_in, half_in],
        out_shape=[yh, yh],
    )(*aggs, *ys, dinv2d, b1_2d, w2)


def _k5_body(a0_ref, a1_ref, y0_ref, y1_ref, dinv_ref, b2_ref, h0_ref,
             out_ref):
    d = dinv_ref[...]
    oa = (a0_ref[...].astype(jnp.float32) + y0_ref[...].astype(jnp.float32)) * d
    ob = (a1_ref[...].astype(jnp.float32) + y1_ref[...].astype(jnp.float32)) * d
    out_ref[...] = jnp.concatenate([oa, ob], axis=1) + b2_ref[...] + h0_ref[...]


def _k5(aggs, ys, dinv2d, b2_2d, h0):
    blk = 1000
    grid = (NUM_NODES // blk,)
    half_in = pl.BlockSpec((blk, HALF), lambda i: (i, 0))
    return pl.pallas_call(
        _k5_body,
        grid=grid,
        in_specs=[half_in, half_in, half_in, half_in,
                  pl.BlockSpec((blk, 1), lambda i: (i, 0)),
                  pl.BlockSpec((1, EMBED), lambda i: (0, 0)),
                  pl.BlockSpec((blk, EMBED), lambda i: (i, 0))],
        out_specs=pl.BlockSpec((blk, EMBED), lambda i: (i, 0)),
        out_shape=jax.ShapeDtypeStruct((NUM_NODES, EMBED), jnp.float32),
    )(*aggs, *ys, dinv2d, b2_2d, h0)


def _sc_agg_body(y0_hbm, y1_hbm, src_hbm, dst_hbm, out0_hbm, out1_hbm,
                 idx_v, rr_v, g0, g1, s0, s1, agg_sh):
    c = lax.axis_index("c")
    s = lax.axis_index("s")
    base = s * SLAB

    pltpu.sync_copy(src_hbm.at[s], idx_v.at[0])
    pltpu.sync_copy(dst_hbm.at[s], idx_v.at[1])

    def _zero_row(i, carry):
        for j in range(HALF // 32):
            rr_v[0, i, pl.ds(j * 32, 32)] = jnp.zeros((32,), BF)
        return carry
    lax.fori_loop(0, ZCH, _zero_row, 0)
    for i in range(SLAB // ZCH):
        pltpu.sync_copy(rr_v.at[0, pl.ds(0, ZCH), :],
                        agg_sh.at[pl.ds(base + i * ZCH, ZCH), :])

    @pl.when(s == N_SUB - 1)
    def _zero_tail():
        pltpu.sync_copy(rr_v.at[0, pl.ds(0, 24), :],
                        agg_sh.at[pl.ds(N_SUB * SLAB, 24), :])

    plsc.subcore_barrier()

    def _run(y_ref):
        rbuf = rr_v.at[0]
        rbuf1 = rr_v.at[1]

        def gather(b, r, sem):
            pltpu.async_copy(y_ref.at[idx_v.at[0, b]], r, sem)

        def scatter(b, r, sem):
            pltpu.async_copy(r, agg_sh.at[idx_v.at[1, b]], sem, add=True)

        def wait_gather(r, sem):
            pltpu.make_async_copy(y_ref.at[idx_v.at[0, 0]], r, sem).wait()

        def wait_scatter(r, sem):
            pltpu.make_async_copy(r, agg_sh.at[idx_v.at[1, 0]], sem).wait()

        gather(0, rbuf, g0)
        gather(1, rbuf1, g1)

        def body(p, carry):
            b = 2 * p
            wait_gather(rbuf, g0)
            scatter(b, rbuf, s0)
            wait_gather(rbuf1, g1)
            scatter(b + 1, rbuf1, s1)

            @pl.when(b + 2 < N_BATCH)
            def _next0():
                wait_scatter(rbuf, s0)
                gather(b + 2, rbuf, g0)

            @pl.when(b + 3 < N_BATCH)
            def _next1():
                wait_scatter(rbuf1, s1)
                gather(b + 3, rbuf1, g1)
            return carry
        lax.fori_loop(0, N_BATCH // 2, body, 0)
        if N_BATCH % 2:
            wait_gather(rbuf, g0)
            scatter(N_BATCH - 1, rbuf, s0)
        wait_scatter(rbuf, s0)
        wait_scatter(rbuf1, s1)

    pl.when(c == 0)(lambda: _run(y0_hbm))
    pl.when(c == 1)(lambda: _run(y1_hbm))
    plsc.subcore_barrier()

    def _writeback(out_ref):
        pltpu.sync_copy(agg_sh.at[pl.ds(base, SLAB), :],
                        out_ref.at[pl.ds(base, SLAB), :])

        @pl.when(s == N_SUB - 1)
        def _tail():
            pltpu.sync_copy(agg_sh.at[pl.ds(N_SUB * SLAB, 16), :],
                            out_ref.at[pl.ds(N_SUB * SLAB, 16), :])

    pl.when(c == 0)(lambda: _writeback(out0_hbm))
    pl.when(c == 1)(lambda: _writeback(out1_hbm))


def _sc_agg(ys, src3, dst3):
    mesh = plsc.VectorSubcoreMesh(core_axis_name="c", subcore_axis_name="s")
    yh = jax.ShapeDtypeStruct((NUM_NODES, HALF), BF)
    return pl.kernel(
        _sc_agg_body,
        out_type=[yh, yh],
        mesh=mesh,
        compiler_params=pltpu.CompilerParams(use_tc_tiling_on_sc=False),
        scratch_types=[
            pltpu.VMEM((2, N_BATCH, BATCH), jnp.int32),
            pltpu.VMEM((2, BATCH, HALF), BF),
            pltpu.SemaphoreType.DMA,
            pltpu.SemaphoreType.DMA,
            pltpu.SemaphoreType.DMA,
            pltpu.SemaphoreType.DMA,
            pltpu.VMEM_SHARED((ACC_ROWS, HALF), BF),
        ],
    )(*ys, src3, dst3)


def kernel(node_ids, edge_index, W_emb, b_emb, W1, b1, W2, b2):
    src = edge_index[0]
    dst = edge_index[1]
    pad_src = (jnp.arange(PAD, dtype=jnp.int32) * 89) % NUM_NODES
    pad_dst = NUM_NODES + (jnp.arange(PAD, dtype=jnp.int32) % 8)
    src3 = jnp.concatenate(
        [src.reshape(N_SUB, EDGES_PER_TILE),
         jnp.broadcast_to(pad_src, (N_SUB, PAD))], axis=1
    ).reshape(N_SUB, N_BATCH, BATCH)
    dst3 = jnp.concatenate(
        [dst.reshape(N_SUB, EDGES_PER_TILE),
         jnp.broadcast_to(pad_dst, (N_SUB, PAD))], axis=1
    ).reshape(N_SUB, N_BATCH, BATCH)

    deg = jnp.ones((NUM_NODES,), jnp.float32).at[dst].add(1.0)
    dinv2d = lax.rsqrt(deg)[:, None]

    h0 = _k1a(node_ids, W_emb, b_emb[None, :])
    y1_0, y1_1 = _k1b(h0, W1, dinv2d)
    a1 = _sc_agg((y1_0, y1_1), src3, dst3)
    y2 = _k3(a1, (y1_0, y1_1), dinv2d, b1[None, :], W2)
    a2 = _sc_agg(y2, src3, dst3)
    return _k5(a2, y2, dinv2d, b2[None, :], h0)
